# baseline (speedup 1.0000x reference)
import functools

import jax
import jax.numpy as jnp
from jax import lax
from jax.experimental import pallas as pl
from jax.experimental.pallas import tpu as pltpu
from jax.experimental.pallas import tpu_sc as plsc

NUM_CORES = 2
NUM_SUBCORES = 16
NW = NUM_CORES * NUM_SUBCORES

CHUNK = 128
MACRO = 4
NBUF = 2


@functools.lru_cache(maxsize=None)
def _make_kernel(B, D):
    b_per_w = B // NW
    n_chunk = b_per_w // CHUNK
    n_macro = n_chunk // MACRO
    n_outer = n_macro // NBUF
    rows_per_macro = MACRO * CHUNK

    mesh = plsc.VectorSubcoreMesh(core_axis_name="c", subcore_axis_name="s")

    @functools.partial(
        pl.kernel,
        out_type=jax.ShapeDtypeStruct((B, D), jnp.float32),
        mesh=mesh,
        scratch_types=[
            pltpu.VMEM((n_chunk, CHUNK), jnp.int32),
            pltpu.VMEM((NBUF, rows_per_macro, D), jnp.float32),
            pltpu.SemaphoreType.DMA((NBUF,)),
            pltpu.SemaphoreType.DMA((NBUF,)),
        ],
        compiler_params=pltpu.CompilerParams(use_tc_tiling_on_sc=False),
    )
    def k(idx_hbm, table_hbm, out_hbm, idx_v, rows_v, gsem, osem):
        wid = lax.axis_index("s") * NUM_CORES + lax.axis_index("c")
        base = wid * b_per_w

        pltpu.sync_copy(idx_hbm.at[wid], idx_v)

        def fire(m, b):
            for j in range(MACRO):
                c = m * MACRO + j
                pltpu.async_copy(
                    table_hbm.at[idx_v.at[c]],
                    rows_v.at[b, pl.ds(j * CHUNK, CHUNK)],
                    gsem.at[b],
                )

        def drain(b):
            pltpu.make_async_copy(
                table_hbm.at[pl.ds(0, rows_per_macro)],
                rows_v.at[b],
                gsem.at[b],
            ).wait()

        def out_slice(m):
            return out_hbm.at[pl.ds(base + m * rows_per_macro, rows_per_macro)]

        fire(0, 0)

        @pl.loop(0, n_outer)
        def _(g):
            for b in range(NBUF):
                m = g * NBUF + b
                drain(b)
                pltpu.async_copy(rows_v.at[b], out_slice(m), osem.at[b])

                b1 = (b + 1) % NBUF

                @pl.when(m + 1 < n_macro)
                def _fire_next():
                    @pl.when(m + 1 >= NBUF)
                    def _wait_prev_out():
                        pltpu.make_async_copy(
                            rows_v.at[b1], out_slice(m), osem.at[b1]
                        ).wait()

                    fire(m + 1, b1)

        for b in range(NBUF):
            pltpu.make_async_copy(
                rows_v.at[b], out_slice(0), osem.at[b]
            ).wait()

    return k


def kernel(indices, table):
    batch, fields = indices.shape
    n_rows, feats = table.shape
    B = batch * fields
    idx = indices.reshape(-1).astype(jnp.int32)
    idx3 = idx.reshape(NW, B // NW // CHUNK, CHUNK)
    out = _make_kernel(B, feats)(idx3, table)
    return out.reshape(batch, fields, feats)


# speedup vs baseline: 1.0134x; 1.0134x over previous
import functools

import jax
import jax.numpy as jnp
from jax import lax
from jax.experimental import pallas as pl
from jax.experimental.pallas import tpu as pltpu
from jax.experimental.pallas import tpu_sc as plsc

NUM_CORES = 2
NUM_SUBCORES = 16
NW = NUM_CORES * NUM_SUBCORES

CHUNK = 128
MACRO = 13
NBUF = 2


@functools.lru_cache(maxsize=None)
def _make_kernel(B, D):
    b_per_w = B // NW
    n_chunk = b_per_w // CHUNK
    n_macro = n_chunk // MACRO
    n_outer = n_macro // NBUF
    rows_per_macro = MACRO * CHUNK

    mesh = plsc.VectorSubcoreMesh(core_axis_name="c", subcore_axis_name="s")

    @functools.partial(
        pl.kernel,
        out_type=jax.ShapeDtypeStruct((B, D), jnp.float32),
        mesh=mesh,
        scratch_types=[
            pltpu.VMEM((n_chunk, CHUNK), jnp.int32),
            pltpu.VMEM((NBUF, rows_per_macro, D), jnp.float32),
            pltpu.SemaphoreType.DMA((NBUF,)),
            pltpu.SemaphoreType.DMA((NBUF,)),
        ],
        compiler_params=pltpu.CompilerParams(use_tc_tiling_on_sc=False),
    )
    def k(idx_hbm, table_hbm, out_hbm, idx_v, rows_v, gsem, osem):
        wid = lax.axis_index("s") * NUM_CORES + lax.axis_index("c")
        base = wid * b_per_w

        pltpu.sync_copy(idx_hbm.at[wid], idx_v)

        def fire(m, b):
            for j in range(MACRO):
                c = m * MACRO + j
                pltpu.async_copy(
                    table_hbm.at[idx_v.at[c]],
                    rows_v.at[b, pl.ds(j * CHUNK, CHUNK)],
                    gsem.at[b],
                )

        def drain(b):
            pltpu.make_async_copy(
                table_hbm.at[pl.ds(0, rows_per_macro)],
                rows_v.at[b],
                gsem.at[b],
            ).wait()

        def out_slice(m):
            return out_hbm.at[pl.ds(base + m * rows_per_macro, rows_per_macro)]

        for b in range(NBUF):
            fire(b, b)

        @pl.loop(0, n_macro)
        def _(m):
            b = m % NBUF
            drain(b)
            cp = pltpu.async_copy(rows_v.at[b], out_slice(m), osem.at[b])
            cp.wait()

            @pl.when(m + NBUF < n_macro)
            def _fire_next():
                fire(m + NBUF, b)

    return k


def kernel(indices, table):
    batch, fields = indices.shape
    n_rows, feats = table.shape
    B = batch * fields
    idx = indices.reshape(-1).astype(jnp.int32)
    idx3 = idx.reshape(NW, B // NW // CHUNK, CHUNK)
    out = _make_kernel(B, feats)(idx3, table)
    return out.reshape(batch, fields, feats)
